# baseline (speedup 1.0000x reference)
import jax
import jax.numpy as jnp
import numpy as np
from jax.experimental import pallas as pl

D = 128
NSIGMA = 5

_INTERP = False


def _rowblk(n):
    for r in (1000, 1024, 500, 512, 250, 200, 125, 100, 8):
        if n % r == 0:
            return r
    return n


def _erf(x):
    p = 0.3275911
    a1, a2, a3, a4, a5 = (0.254829592, -0.284496736, 1.421413741,
                          -1.453152027, 1.061405429)
    s = jnp.sign(x)
    ax = jnp.abs(x)
    t = 1.0 / (1.0 + p * ax)
    poly = ((((a5 * t + a4) * t + a3) * t + a2) * t + a1) * t
    return s * (1.0 - poly * jnp.exp(-ax * ax))


def _film_math(x, c, W1, b1, W2, b2, W3, b3):
    h = jnp.maximum(jnp.dot(c, W1, preferred_element_type=jnp.float32) + b1, 0.0)
    h = jnp.maximum(jnp.dot(h, W2, preferred_element_type=jnp.float32) + b2, 0.0)
    gb = jnp.dot(h, W3, preferred_element_type=jnp.float32) + b3
    return gb[:, :D] * x + gb[:, D:]


def _full(shape):
    nd = len(shape)
    return pl.BlockSpec(shape, lambda i, _nd=nd: (0,) * _nd)


def _rows(r, shape):
    rest = shape[1:]
    nd = len(shape)
    return pl.BlockSpec((r,) + rest, lambda i, _nd=nd: (i,) + (0,) * (_nd - 1))


def _call(body, grid, in_specs, out_specs, out_shape, args):
    return pl.pallas_call(
        body, grid=grid, in_specs=in_specs, out_specs=out_specs,
        out_shape=out_shape, interpret=_INTERP)(*args)


def _dop_impl(t_ref, c_ref, g_ref, tbl_ref,
              dW1, db1, dW2, db2, dW3, db3,
              cW1, cb1, cW2, cb2, cW3, cb3, o_ref):
    t = t_ref[...]
    iota = jax.lax.broadcasted_iota(jnp.int32, (t.shape[0], 3), 1).astype(jnp.float32)
    oh = (t == iota).astype(jnp.float32)
    x = jnp.dot(oh, tbl_ref[...], preferred_element_type=jnp.float32)
    x = _film_math(x, c_ref[...], dW1[...], db1[...], dW2[...], db2[...],
                   dW3[...], db3[...])
    x = _film_math(x, g_ref[...], cW1[...], cb1[...], cW2[...], cb2[...],
                   cW3[...], cb3[...])
    o_ref[...] = x


def _dopant_features(types_f, conc, geom, tbl, fd, fc):
    n = types_f.shape[0]
    r = _rowblk(n)
    args = [types_f, conc, geom, tbl] + list(fd) + list(fc)
    specs = [_rows(r, types_f.shape), _rows(r, conc.shape),
             _rows(r, geom.shape), _full(tbl.shape)]
    specs += [_full(a.shape) for a in list(fd) + list(fc)]
    return _call(_dop_impl, (n // r,), specs, _rows(r, (n, D)),
                 jax.ShapeDtypeStruct((n, D), jnp.float32), args)


def _ii_impl(nr_ref, nc_ref, o_ref):
    nr = nr_ref[...]
    s = 1.0 + jax.lax.broadcasted_iota(jnp.int32, (1, NSIGMA), 1).astype(jnp.float32)
    r1i, r1o = nr[:, 0:1], nr[:, 1:2]
    r2i, r2o = nr[:, 2:3], nr[:, 3:4]
    c1 = 0.5 * (r1i + r1o)
    c2 = 0.5 * (r2i + r2o)
    sq2 = np.float32(np.sqrt(2.0))
    a = _erf((r1o - r2i) / (s * sq2))
    b = _erf((r1i - r2o) / (s * sq2))
    g = jnp.exp(-((c1 - c2) ** 2) / (2.0 * s * s))
    nc = nc_ref[...]
    o_ref[...] = (a - b) * g * (nc[:, 0:1] * nc[:, 1:2])


def _integrated(nr, nc):
    n = nr.shape[0]
    r = _rowblk(n)
    return _call(_ii_impl, (n // r,),
                 [_rows(r, nr.shape), _rows(r, nc.shape)],
                 _rows(r, (n, NSIGMA)),
                 jax.ShapeDtypeStruct((n, NSIGMA), jnp.float32), [nr, nc])


def _emb_impl(ti_ref, ii_ref, sc_ref, sh_ref, We_ref, be_ref,
              W1, b1, W2, b2, W3, b3, o_ref):
    base = jnp.dot(ti_ref[...], We_ref[...],
                   preferred_element_type=jnp.float32) + be_ref[...]
    cond = ii_ref[...] * sc_ref[...] + sh_ref[...]
    o_ref[...] = _film_math(base, cond, W1[...], b1[...], W2[...], b2[...],
                            W3[...], b3[...])


def _embed_film(ti_f, ii, scale, shift, We, be, fw):
    n = ti_f.shape[0]
    r = _rowblk(n)
    args = [ti_f, ii, scale, shift, We, be] + list(fw)
    specs = [_rows(r, ti_f.shape), _rows(r, ii.shape), _full(scale.shape),
             _full(shift.shape), _full(We.shape), _full(be.shape)]
    specs += [_full(a.shape) for a in fw]
    return _call(_emb_impl, (n // r,), specs, _rows(r, (n, D)),
                 jax.ShapeDtypeStruct((n, D), jnp.float32), args)


def _region_map(cuts):
    def index_map(i):
        reg = jnp.int32(0)
        for c in cuts:
            reg = reg + jnp.where(i >= c, 1, 0).astype(jnp.int32)
        return (reg, 0, 0)
    return index_map


def _mmsel_impl(x_ref, w_ref, b_ref, o_ref):
    o_ref[...] = jnp.dot(x_ref[...], w_ref[0],
                         preferred_element_type=jnp.float32) + b_ref[0]


def _mm_sel(x, w_stack, b_stack, cuts):
    n = x.shape[0]
    r = _rowblk(n)
    return _call(
        _mmsel_impl, (n // r,),
        [_rows(r, x.shape),
         pl.BlockSpec((1,) + w_stack.shape[1:], _region_map(cuts)),
         pl.BlockSpec((1,) + b_stack.shape[1:], _region_map(cuts))],
        _rows(r, (n, w_stack.shape[2])),
        jax.ShapeDtypeStruct((n, w_stack.shape[2]), jnp.float32),
        [x, w_stack, b_stack])


def _e_impl(a_ref, b_ref, att_ref, o_ref):
    z = a_ref[...] + b_ref[...]
    z = jnp.where(z > 0, z, 0.2 * z)
    e = jnp.dot(z, att_ref[0], preferred_element_type=jnp.float32)
    o_ref[...] = jnp.exp(e)


def _edge_e(hl_s, hr_d, att_stack, cuts):
    n = hl_s.shape[0]
    r = _rowblk(n)
    return _call(
        _e_impl, (n // r,),
        [_rows(r, hl_s.shape), _rows(r, hr_d.shape),
         pl.BlockSpec((1,) + att_stack.shape[1:], _region_map(cuts))],
        _rows(r, (n, 1)),
        jax.ShapeDtypeStruct((n, 1), jnp.float32),
        [hl_s, hr_d, att_stack])


def _wv_impl(ex_ref, h_ref, o_ref):
    ex = ex_ref[...]
    o_ref[:, :D] = ex * h_ref[...]
    o_ref[:, D:] = ex


def _edge_wv(ex, hl_s):
    n = ex.shape[0]
    r = _rowblk(n)
    return _call(_wv_impl, (n // r,),
                 [_rows(r, ex.shape), _rows(r, hl_s.shape)],
                 _rows(r, (n, D + 1)),
                 jax.ShapeDtypeStruct((n, D + 1), jnp.float32),
                 [ex, hl_s])


def _norm129(seg):
    return seg[:, :D] / (seg[:, D:] + 1e-16)


def _silu2_impl(a_ref, b_ref, c_ref, o_ref):
    z = _norm129(a_ref[...]) + _norm129(b_ref[...]) + c_ref[...]
    o_ref[...] = z / (1.0 + jnp.exp(-z))


def _silu2(seg, off1, off2, n, bias_sum):
    r = _rowblk(n)
    o1 = off1 // r
    o2 = off2 // r
    return _call(_silu2_impl, (n // r,),
                 [pl.BlockSpec((r, seg.shape[1]), lambda i: (i + o1, 0)),
                  pl.BlockSpec((r, seg.shape[1]), lambda i: (i + o2, 0)),
                  _full(bias_sum.shape)],
                 _rows(r, (n, D)),
                 jax.ShapeDtypeStruct((n, D), jnp.float32),
                 [seg, seg, bias_sum])


def _silu1_impl(a_ref, c_ref, o_ref):
    z = _norm129(a_ref[...]) + c_ref[...]
    o_ref[...] = z / (1.0 + jnp.exp(-z))


def _silu1(seg, off, n, bias):
    r = _rowblk(n)
    o1 = off // r
    return _call(_silu1_impl, (n // r,),
                 [pl.BlockSpec((r, seg.shape[1]), lambda i: (i + o1, 0)),
                  _full(bias.shape)],
                 _rows(r, (n, D)),
                 jax.ShapeDtypeStruct((n, D), jnp.float32), [seg, bias])


def _gseg_impl(ids_ref, x_ref, o_ref):
    @pl.when(pl.program_id(0) == 0)
    def _():
        o_ref[...] = jnp.zeros_like(o_ref)

    ids = ids_ref[0, 0, :]
    n_graph = o_ref.shape[0]
    iota = jax.lax.broadcasted_iota(jnp.int32, (n_graph, ids.shape[0]), 0)
    oh = (iota == ids[None, :]).astype(jnp.float32)
    o_ref[...] += jnp.dot(oh, x_ref[...], preferred_element_type=jnp.float32)


def _graph_segsum(ids, x, n_graph):
    n = x.shape[0]
    r = _rowblk(n)
    ids3 = ids.reshape(n // r, 1, r)
    return pl.pallas_call(
        _gseg_impl, grid=(n // r,),
        in_specs=[pl.BlockSpec((1, 1, r), lambda i: (i, 0, 0)),
                  _rows(r, x.shape)],
        out_specs=pl.BlockSpec((n_graph, D), lambda i: (0, 0)),
        out_shape=jax.ShapeDtypeStruct((n_graph, D), jnp.float32),
        interpret=_INTERP)(ids3, x)


def _gat_layer(x_d, x_i, x_a, idx, Wl, Wr, at, bb,
               d_scale=None, d_shift=None):
    nd, ni, na = x_d.shape[0], x_i.shape[0], x_a.shape[0]
    x_src_all = jnp.concatenate([x_d, x_i, x_d, x_a], 0)
    x_dst_all = jnp.concatenate([x_i, x_d, x_a, x_d], 0)
    z1 = jnp.zeros((1, D), jnp.float32)
    if d_scale is None:
        Wl_s, bl_s = Wl, jnp.zeros((4, 1, D), jnp.float32)
        Wr_s, br_s = Wr, jnp.zeros((4, 1, D), jnp.float32)
    else:
        def fold(W):
            return d_scale[:, None] * W

        def fbias(W):
            return (d_shift @ W)[None, :]

        Wl_s = jnp.stack([fold(Wl[0]), Wl[1], fold(Wl[2]), Wl[3]])
        bl_s = jnp.stack([fbias(Wl[0]), z1, fbias(Wl[2]), z1])
        Wr_s = jnp.stack([Wr[0], fold(Wr[1]), Wr[2], fold(Wr[3])])
        br_s = jnp.stack([z1, fbias(Wr[1]), z1, fbias(Wr[3])])
    Hl = _mm_sel(x_src_all, Wl_s, bl_s, idx["cuts_src"])
    Hr = _mm_sel(x_dst_all, Wr_s, br_s, idx["cuts_dst"])
    hl_s = jnp.take(Hl, idx["src_comb"], axis=0)
    hr_d = jnp.take(Hr, idx["dst_comb"], axis=0)
    ex = _edge_e(hl_s, hr_d, at[:, :, None], idx["cuts_edge"])
    wv = _edge_wv(ex, hl_s)
    seg = jax.ops.segment_sum(wv, idx["dst_comb"],
                              num_segments=ni + 2 * nd + na)
    x_i2 = _silu1(seg, 0, ni, bb[0][None, :])
    x_a2 = _silu1(seg, ni + nd, na, bb[2][None, :])
    x_d2 = _silu2(seg, ni, ni + nd + na, nd, (bb[1] + bb[3])[None, :])
    return x_d2, x_i2, x_a2


def kernel(dopant_types, dopant_concs, dopant_constraint_indices, interaction_type_indices, interaction_types, interaction_dopant_indices, intraaction_type_indices, intraaction_types, intraaction_dopant_indices, ei_d2i_src, ei_d2i_dst, ei_i2d_src, ei_i2d_dst, ei_d2a_src, ei_d2a_dst, ei_a2d_src, ei_a2d_dst, radii, constraint_radii_idx, batch_dopant, dop_table, fd_W1, fd_b1, fd_W2, fd_b2, fd_W3, fd_b3, fc_W1, fc_b1, fc_W2, fc_b2, fc_W3, fc_b3, fi_W1, fi_b1, fi_W2, fi_b2, fi_W3, fi_b3, fa_W1, fa_b1, fa_W2, fa_b2, fa_W3, fa_b3, bnd_g, bnd_b, bni_g, bni_b, bna_g, bna_b, Wi_e, bi_e, Wa_e, ba_e, gat_Wl, gat_Wr, gat_att, gat_bias):
    n_dop = dopant_types.shape[0]
    n_int = interaction_types.shape[0]
    n_intra = intraaction_types.shape[0]
    n_graph = 512
    eps = 1e-5

    _radii = radii[constraint_radii_idx]
    geom = _radii[dopant_constraint_indices]
    conc = dopant_concs[:, None]
    types_f = dopant_types.astype(jnp.float32)[:, None]

    def c2(b):
        return b.reshape(1, -1)

    fd = (fd_W1, c2(fd_b1), fd_W2, c2(fd_b2), fd_W3, c2(fd_b3))
    fc = (fc_W1, c2(fc_b1), fc_W2, c2(fc_b2), fc_W3, c2(fc_b3))
    fi = (fi_W1, c2(fi_b1), fi_W2, c2(fi_b2), fi_W3, c2(fi_b3))
    fa = (fa_W1, c2(fa_b1), fa_W2, c2(fa_b2), fa_W3, c2(fa_b3))

    x_d_raw = _dopant_features(types_f, conc, geom, dop_table, fd, fc)
    mu = x_d_raw.mean(0)
    var = x_d_raw.var(0)
    d_scale = bnd_g / jnp.sqrt(var + eps)
    d_shift = bnd_b - mu * d_scale

    dftab = jnp.concatenate([geom, conc], 1)
    pair_idx = jnp.concatenate([interaction_dopant_indices,
                                intraaction_dopant_indices], 0)
    gpair = dftab[pair_idx]
    g_i = gpair[:n_int]
    g_a = gpair[n_int:]
    nr_i = g_i[:, :, :2].reshape(-1, 4)
    nc_i = g_i[:, :, 2]
    ii = _integrated(nr_i, nc_i)
    mu_i = ii.mean(0)
    var_i = ii.var(0)
    sc_i = (bni_g / jnp.sqrt(var_i + eps))[None, :]
    sh_i = (bni_b - mu_i * sc_i[0])[None, :]
    x_i = _embed_film(interaction_type_indices.astype(jnp.float32), ii,
                      sc_i, sh_i, Wi_e, c2(bi_e), fi)

    nr_a = g_a[:, :, :2].reshape(-1, 4)
    nc_a = g_a[:, :, 2]
    ia = _integrated(nr_a, nc_a)
    mu_a = ia.mean(0)
    var_a = ia.var(0)
    sc_a = (bna_g / jnp.sqrt(var_a + eps))[None, :]
    sh_a = (bna_b - mu_a * sc_a[0])[None, :]
    x_a = _embed_film(intraaction_type_indices.astype(jnp.float32), ia,
                      sc_a, sh_a, Wa_e, c2(ba_e), fa)

    nd, ni, na = n_dop, n_int, n_intra
    ne = ei_d2i_src.shape[0]
    src_comb = jnp.concatenate([
        ei_d2i_src, ei_i2d_src + nd, ei_d2a_src + (nd + ni),
        ei_a2d_src + (nd + ni + nd)])
    dst_comb = jnp.concatenate([
        ei_d2i_dst, ei_i2d_dst + ni, ei_d2a_dst + (ni + nd),
        ei_a2d_dst + (ni + nd + na)])
    r_src = _rowblk(2 * nd + ni + na)
    r_dst = _rowblk(ni + 2 * nd + na)
    r_e = _rowblk(4 * ne)
    idx = {
        "src_comb": src_comb,
        "dst_comb": dst_comb,
        "cuts_src": (nd // r_src, (nd + ni) // r_src,
                     (nd + ni + nd) // r_src),
        "cuts_dst": (ni // r_dst, (ni + nd) // r_dst,
                     (ni + nd + na) // r_dst),
        "cuts_edge": (ne // r_e, 2 * ne // r_e, 3 * ne // r_e),
    }

    x_d = x_d_raw
    for l in range(3):
        ss = d_scale if l == 0 else None
        sh = d_shift if l == 0 else None
        x_d, x_i, x_a = _gat_layer(x_d, x_i, x_a, idx, gat_Wl[l], gat_Wr[l],
                                   gat_att[l], gat_bias[l],
                                   d_scale=ss, d_shift=sh)

    return _graph_segsum(batch_dopant, x_d, n_graph)


# speedup vs baseline: 1.1022x; 1.1022x over previous
import jax
import jax.numpy as jnp
import numpy as np
from jax.experimental import pallas as pl

D = 128
NSIGMA = 5

_INTERP = False


def _rowblk(n):
    for r in (1000, 1024, 500, 512, 250, 200, 125, 100, 8):
        if n % r == 0:
            return r
    return n


def _erf(x):
    p = 0.3275911
    a1, a2, a3, a4, a5 = (0.254829592, -0.284496736, 1.421413741,
                          -1.453152027, 1.061405429)
    s = jnp.sign(x)
    ax = jnp.abs(x)
    t = 1.0 / (1.0 + p * ax)
    poly = ((((a5 * t + a4) * t + a3) * t + a2) * t + a1) * t
    return s * (1.0 - poly * jnp.exp(-ax * ax))


def _film_math(x, c, W1, b1, W2, b2, W3, b3):
    h = jnp.maximum(jnp.dot(c, W1, preferred_element_type=jnp.float32) + b1, 0.0)
    h = jnp.maximum(jnp.dot(h, W2, preferred_element_type=jnp.float32) + b2, 0.0)
    gb = jnp.dot(h, W3, preferred_element_type=jnp.float32) + b3
    return gb[:, :D] * x + gb[:, D:]


def _full(shape):
    nd = len(shape)
    return pl.BlockSpec(shape, lambda i, _nd=nd: (0,) * _nd)


def _rows(r, shape):
    rest = shape[1:]
    nd = len(shape)
    return pl.BlockSpec((r,) + rest, lambda i, _nd=nd: (i,) + (0,) * (_nd - 1))


def _call(body, grid, in_specs, out_specs, out_shape, args):
    return pl.pallas_call(
        body, grid=grid, in_specs=in_specs, out_specs=out_specs,
        out_shape=out_shape, interpret=_INTERP)(*args)


def _dop_impl(t_ref, c_ref, g_ref, tbl_ref,
              dW1, db1, dW2, db2, dW3, db3,
              cW1, cb1, cW2, cb2, cW3, cb3, o_ref):
    t = t_ref[...]
    iota = jax.lax.broadcasted_iota(jnp.int32, (t.shape[0], 3), 1).astype(jnp.float32)
    oh = (t == iota).astype(jnp.float32)
    x = jnp.dot(oh, tbl_ref[...], preferred_element_type=jnp.float32)
    x = _film_math(x, c_ref[...], dW1[...], db1[...], dW2[...], db2[...],
                   dW3[...], db3[...])
    x = _film_math(x, g_ref[...], cW1[...], cb1[...], cW2[...], cb2[...],
                   cW3[...], cb3[...])
    o_ref[...] = x


def _dopant_features(types_f, conc, geom, tbl, fd, fc):
    n = types_f.shape[0]
    r = _rowblk(n)
    args = [types_f, conc, geom, tbl] + list(fd) + list(fc)
    specs = [_rows(r, types_f.shape), _rows(r, conc.shape),
             _rows(r, geom.shape), _full(tbl.shape)]
    specs += [_full(a.shape) for a in list(fd) + list(fc)]
    return _call(_dop_impl, (n // r,), specs, _rows(r, (n, D)),
                 jax.ShapeDtypeStruct((n, D), jnp.float32), args)


def _ii_impl(nr_ref, nc_ref, o_ref):
    nr = nr_ref[...]
    s = 1.0 + jax.lax.broadcasted_iota(jnp.int32, (1, NSIGMA), 1).astype(jnp.float32)
    r1i, r1o = nr[:, 0:1], nr[:, 1:2]
    r2i, r2o = nr[:, 2:3], nr[:, 3:4]
    c1 = 0.5 * (r1i + r1o)
    c2 = 0.5 * (r2i + r2o)
    sq2 = np.float32(np.sqrt(2.0))
    a = _erf((r1o - r2i) / (s * sq2))
    b = _erf((r1i - r2o) / (s * sq2))
    g = jnp.exp(-((c1 - c2) ** 2) / (2.0 * s * s))
    nc = nc_ref[...]
    o_ref[...] = (a - b) * g * (nc[:, 0:1] * nc[:, 1:2])


def _integrated(nr, nc):
    n = nr.shape[0]
    r = _rowblk(n)
    return _call(_ii_impl, (n // r,),
                 [_rows(r, nr.shape), _rows(r, nc.shape)],
                 _rows(r, (n, NSIGMA)),
                 jax.ShapeDtypeStruct((n, NSIGMA), jnp.float32), [nr, nc])


def _emb_impl(ti_ref, ii_ref, sc_ref, sh_ref, We_ref, be_ref,
              W1, b1, W2, b2, W3, b3, o_ref):
    base = jnp.dot(ti_ref[...], We_ref[...],
                   preferred_element_type=jnp.float32) + be_ref[...]
    cond = ii_ref[...] * sc_ref[...] + sh_ref[...]
    o_ref[...] = _film_math(base, cond, W1[...], b1[...], W2[...], b2[...],
                            W3[...], b3[...])


def _embed_film(ti_f, ii, scale, shift, We, be, fw):
    n = ti_f.shape[0]
    r = _rowblk(n)
    args = [ti_f, ii, scale, shift, We, be] + list(fw)
    specs = [_rows(r, ti_f.shape), _rows(r, ii.shape), _full(scale.shape),
             _full(shift.shape), _full(We.shape), _full(be.shape)]
    specs += [_full(a.shape) for a in fw]
    return _call(_emb_impl, (n // r,), specs, _rows(r, (n, D)),
                 jax.ShapeDtypeStruct((n, D), jnp.float32), args)


def _mm_impl(x_ref, w_ref, b_ref, o_ref):
    o_ref[...] = jnp.dot(x_ref[...], w_ref[...],
                         preferred_element_type=jnp.float32) + b_ref[...]


def _mm(x, w, b):
    n = x.shape[0]
    r = _rowblk(n)
    return _call(_mm_impl, (n // r,),
                 [_rows(r, x.shape), _full(w.shape), _full(b.shape)],
                 _rows(r, (n, w.shape[1])),
                 jax.ShapeDtypeStruct((n, w.shape[1]), jnp.float32),
                 [x, w, b])


def _e_impl(a_ref, b_ref, att_ref, o_ref):
    z = a_ref[...] + b_ref[...]
    z = jnp.where(z > 0, z, 0.2 * z)
    e = jnp.dot(z, att_ref[...], preferred_element_type=jnp.float32)
    o_ref[...] = jnp.exp(e)


def _edge_e(hl_s, hr_d, att_col):
    n = hl_s.shape[0]
    r = _rowblk(n)
    return _call(_e_impl, (n // r,),
                 [_rows(r, hl_s.shape), _rows(r, hr_d.shape),
                  _full(att_col.shape)],
                 _rows(r, (n, 1)),
                 jax.ShapeDtypeStruct((n, 1), jnp.float32),
                 [hl_s, hr_d, att_col])


def _wv_impl(ex_ref, h_ref, o_ref):
    ex = ex_ref[...]
    o_ref[:, :D] = ex * h_ref[...]
    o_ref[:, D:] = ex


def _edge_wv(ex, hl_s):
    n = ex.shape[0]
    r = _rowblk(n)
    return _call(_wv_impl, (n // r,),
                 [_rows(r, ex.shape), _rows(r, hl_s.shape)],
                 _rows(r, (n, D + 1)),
                 jax.ShapeDtypeStruct((n, D + 1), jnp.float32),
                 [ex, hl_s])


def _norm129(seg):
    return seg[:, :D] / (seg[:, D:] + 1e-16)


def _silu2_impl(a_ref, b_ref, c_ref, o_ref):
    z = _norm129(a_ref[...]) + _norm129(b_ref[...]) + c_ref[...]
    o_ref[...] = z / (1.0 + jnp.exp(-z))


def _silu2(a, b, bias_sum):
    n = a.shape[0]
    r = _rowblk(n)
    return _call(_silu2_impl, (n // r,),
                 [_rows(r, a.shape), _rows(r, b.shape), _full(bias_sum.shape)],
                 _rows(r, (n, D)),
                 jax.ShapeDtypeStruct((n, D), jnp.float32), [a, b, bias_sum])


def _silu1_impl(a_ref, c_ref, o_ref):
    z = _norm129(a_ref[...]) + c_ref[...]
    o_ref[...] = z / (1.0 + jnp.exp(-z))


def _silu1(a, bias):
    n = a.shape[0]
    r = _rowblk(n)
    return _call(_silu1_impl, (n // r,),
                 [_rows(r, a.shape), _full(bias.shape)],
                 _rows(r, (n, D)),
                 jax.ShapeDtypeStruct((n, D), jnp.float32), [a, bias])


def _gseg_impl(ids_ref, x_ref, o_ref):
    @pl.when(pl.program_id(0) == 0)
    def _():
        o_ref[...] = jnp.zeros_like(o_ref)

    ids = ids_ref[0, 0, :]
    n_graph = o_ref.shape[0]
    iota = jax.lax.broadcasted_iota(jnp.int32, (n_graph, ids.shape[0]), 0)
    oh = (iota == ids[None, :]).astype(jnp.float32)
    o_ref[...] += jnp.dot(oh, x_ref[...], preferred_element_type=jnp.float32)


def _graph_segsum(ids, x, n_graph):
    n = x.shape[0]
    r = _rowblk(n)
    ids3 = ids.reshape(n // r, 1, r)
    return pl.pallas_call(
        _gseg_impl, grid=(n // r,),
        in_specs=[pl.BlockSpec((1, 1, r), lambda i: (i, 0, 0)),
                  _rows(r, x.shape)],
        out_specs=pl.BlockSpec((n_graph, D), lambda i: (0, 0)),
        out_shape=jax.ShapeDtypeStruct((n_graph, D), jnp.float32),
        interpret=_INTERP)(ids3, x)


def _gat(x_src, x_dst, src, dst, Wl, Wr, att, n_dst,
         src_scale=None, src_shift=None, dst_scale=None, dst_shift=None):
    zb = jnp.zeros((1, D), jnp.float32)
    if src_scale is not None:
        bl = (src_shift @ Wl)[None, :]
        Wl = src_scale[:, None] * Wl
    else:
        bl = zb
    if dst_scale is not None:
        br = (dst_shift @ Wr)[None, :]
        Wr = dst_scale[:, None] * Wr
    else:
        br = zb
    hl = _mm(x_src, Wl, bl)
    hr = _mm(x_dst, Wr, br)
    hl_s = jnp.take(hl, src, axis=0)
    hr_d = jnp.take(hr, dst, axis=0)
    ex = _edge_e(hl_s, hr_d, att[:, None])
    wv = _edge_wv(ex, hl_s)
    return jax.ops.segment_sum(wv, dst, num_segments=n_dst)


def kernel(dopant_types, dopant_concs, dopant_constraint_indices, interaction_type_indices, interaction_types, interaction_dopant_indices, intraaction_type_indices, intraaction_types, intraaction_dopant_indices, ei_d2i_src, ei_d2i_dst, ei_i2d_src, ei_i2d_dst, ei_d2a_src, ei_d2a_dst, ei_a2d_src, ei_a2d_dst, radii, constraint_radii_idx, batch_dopant, dop_table, fd_W1, fd_b1, fd_W2, fd_b2, fd_W3, fd_b3, fc_W1, fc_b1, fc_W2, fc_b2, fc_W3, fc_b3, fi_W1, fi_b1, fi_W2, fi_b2, fi_W3, fi_b3, fa_W1, fa_b1, fa_W2, fa_b2, fa_W3, fa_b3, bnd_g, bnd_b, bni_g, bni_b, bna_g, bna_b, Wi_e, bi_e, Wa_e, ba_e, gat_Wl, gat_Wr, gat_att, gat_bias):
    n_dop = dopant_types.shape[0]
    n_int = interaction_types.shape[0]
    n_intra = intraaction_types.shape[0]
    n_graph = 512
    eps = 1e-5

    _radii = radii[constraint_radii_idx]
    geom = _radii[dopant_constraint_indices]
    conc = dopant_concs[:, None]
    types_f = dopant_types.astype(jnp.float32)[:, None]

    def c2(b):
        return b.reshape(1, -1)

    fd = (fd_W1, c2(fd_b1), fd_W2, c2(fd_b2), fd_W3, c2(fd_b3))
    fc = (fc_W1, c2(fc_b1), fc_W2, c2(fc_b2), fc_W3, c2(fc_b3))
    fi = (fi_W1, c2(fi_b1), fi_W2, c2(fi_b2), fi_W3, c2(fi_b3))
    fa = (fa_W1, c2(fa_b1), fa_W2, c2(fa_b2), fa_W3, c2(fa_b3))

    x_d_raw = _dopant_features(types_f, conc, geom, dop_table, fd, fc)
    mu = x_d_raw.mean(0)
    var = x_d_raw.var(0)
    d_scale = bnd_g / jnp.sqrt(var + eps)
    d_shift = bnd_b - mu * d_scale

    dftab = jnp.concatenate([geom, conc], 1)
    pair_idx = jnp.concatenate([interaction_dopant_indices,
                                intraaction_dopant_indices], 0)
    gpair = dftab[pair_idx]
    g_i = gpair[:n_int]
    g_a = gpair[n_int:]
    nr_i = g_i[:, :, :2].reshape(-1, 4)
    nc_i = g_i[:, :, 2]
    ii = _integrated(nr_i, nc_i)
    mu_i = ii.mean(0)
    var_i = ii.var(0)
    sc_i = (bni_g / jnp.sqrt(var_i + eps))[None, :]
    sh_i = (bni_b - mu_i * sc_i[0])[None, :]
    x_i = _embed_film(interaction_type_indices.astype(jnp.float32), ii,
                      sc_i, sh_i, Wi_e, c2(bi_e), fi)

    nr_a = g_a[:, :, :2].reshape(-1, 4)
    nc_a = g_a[:, :, 2]
    ia = _integrated(nr_a, nc_a)
    mu_a = ia.mean(0)
    var_a = ia.var(0)
    sc_a = (bna_g / jnp.sqrt(var_a + eps))[None, :]
    sh_a = (bna_b - mu_a * sc_a[0])[None, :]
    x_a = _embed_film(intraaction_type_indices.astype(jnp.float32), ia,
                      sc_a, sh_a, Wa_e, c2(ba_e), fa)

    x_d = x_d_raw
    for l in range(3):
        Wl = gat_Wl[l]
        Wr = gat_Wr[l]
        at = gat_att[l]
        bb = gat_bias[l]
        if l == 0:
            ss, sh = d_scale, d_shift
        else:
            ss, sh = None, None
        new_i = _gat(x_d, x_i, ei_d2i_src, ei_d2i_dst, Wl[0], Wr[0], at[0],
                     n_int, src_scale=ss, src_shift=sh)
        new_d1 = _gat(x_i, x_d, ei_i2d_src, ei_i2d_dst, Wl[1], Wr[1], at[1],
                      n_dop, dst_scale=ss, dst_shift=sh)
        new_a = _gat(x_d, x_a, ei_d2a_src, ei_d2a_dst, Wl[2], Wr[2], at[2],
                     n_intra, src_scale=ss, src_shift=sh)
        new_d2 = _gat(x_a, x_d, ei_a2d_src, ei_a2d_dst, Wl[3], Wr[3], at[3],
                      n_dop, dst_scale=ss, dst_shift=sh)
        x_d = _silu2(new_d1, new_d2, (bb[1] + bb[3])[None, :])
        x_i = _silu1(new_i, bb[0][None, :])
        x_a = _silu1(new_a, bb[2][None, :])

    return _graph_segsum(batch_dopant, x_d, n_graph)
